# baseline (speedup 1.0000x reference)
import functools

import jax
import jax.numpy as jnp
from jax import lax
from jax.experimental import pallas as pl
from jax.experimental.pallas import tpu as pltpu
from jax.experimental.pallas import tpu_sc as plsc

NUM_EXPERTS = 64
HIDDEN = 768
D_FF = 768
ALPHA = 1.702
LIMIT = 7.0
TOP_K = 2

TILE = 64


def _mlp_body(te_ref, nv_ref, x_ref, guw_ref, gub_ref, dnw_ref, dnb_ref,
              se_ref, so_ref, y_ref, wbuf, bbuf, dbuf, ebuf, sems, slot_ref):
    t = pl.program_id(0)
    nt = pl.num_programs(0)

    def fetch(e, s, wait):
        cps = [
            pltpu.make_async_copy(guw_ref.at[e], wbuf.at[s], sems.at[s, 0]),
            pltpu.make_async_copy(gub_ref.at[e], bbuf.at[s], sems.at[s, 1]),
            pltpu.make_async_copy(dnw_ref.at[e], dbuf.at[s], sems.at[s, 2]),
            pltpu.make_async_copy(dnb_ref.at[e], ebuf.at[s], sems.at[s, 3]),
        ]
        for cp in cps:
            if wait:
                cp.wait()
            else:
                cp.start()

    cur = te_ref[t]
    prev = jnp.where(t > 0, te_ref[jnp.maximum(t - 1, 0)], -1)
    nxt = jnp.where(t + 1 < nt, te_ref[jnp.minimum(t + 1, nt - 1)], cur)
    changed_in = cur != prev

    @pl.when(t == 0)
    def _():
        slot_ref[0] = 0
        fetch(cur, 0, wait=False)

    @pl.when(changed_in & (t > 0))
    def _():
        slot_ref[0] = 1 - slot_ref[0]

    s = slot_ref[0]

    @pl.when(changed_in)
    def _():
        fetch(cur, s, wait=True)

    @pl.when(nxt != cur)
    def _():
        fetch(nxt, 1 - s, wait=False)

    @pl.when(t * TILE < nv_ref[0])
    def _():
        x = x_ref[...]
        gu = lax.dot_general(x, wbuf[s], (((1,), (0,)), ((), ())),
                             preferred_element_type=jnp.float32)
        a = (gu + bbuf[s]).astype(jnp.bfloat16)
        gate = lax.dot_general(a, se_ref[...], (((1,), (0,)), ((), ())),
                               preferred_element_type=jnp.float32)
        up = lax.dot_general(a, so_ref[...], (((1,), (0,)), ((), ())),
                             preferred_element_type=jnp.float32)
        gate = jnp.minimum(gate, LIMIT)
        up = jnp.clip(up, -LIMIT, LIMIT)
        glu = gate * jax.nn.sigmoid(gate * ALPHA)
        h = (up + 1.0) * glu
        y = lax.dot_general(h, dbuf[s], (((1,), (0,)), ((), ())),
                            preferred_element_type=jnp.float32)
        y_ref[...] = y + ebuf[s]


def _grouped_mlp(x_pad, tile_expert, n_valid, gate_up_proj, gate_up_proj_bias,
                 down_proj, down_proj_bias, num_tiles):
    hbm = pl.BlockSpec(memory_space=pltpu.MemorySpace.HBM)
    grid_spec = pltpu.PrefetchScalarGridSpec(
        num_scalar_prefetch=2,
        grid=(num_tiles,),
        in_specs=[
            pl.BlockSpec((TILE, HIDDEN), lambda t, te, nv: (t, 0)),
            hbm, hbm, hbm, hbm,
            pl.BlockSpec((2 * D_FF, D_FF), lambda t, te, nv: (0, 0)),
            pl.BlockSpec((2 * D_FF, D_FF), lambda t, te, nv: (0, 0)),
        ],
        out_specs=pl.BlockSpec((TILE, HIDDEN), lambda t, te, nv: (t, 0)),
        scratch_shapes=[
            pltpu.VMEM((2, HIDDEN, 2 * D_FF), jnp.float32),
            pltpu.VMEM((2, 1, 2 * D_FF), jnp.float32),
            pltpu.VMEM((2, D_FF, HIDDEN), jnp.float32),
            pltpu.VMEM((2, 1, HIDDEN), jnp.float32),
            pltpu.SemaphoreType.DMA((2, 4)),
            pltpu.SMEM((1,), jnp.int32),
        ],
    )
    ff = jnp.arange(D_FF, dtype=jnp.int32)
    col = jnp.arange(2 * D_FF, dtype=jnp.int32)
    sel_even = (col[:, None] == 2 * ff[None, :]).astype(jnp.bfloat16)
    sel_odd = (col[:, None] == 2 * ff[None, :] + 1).astype(jnp.bfloat16)
    return pl.pallas_call(
        _mlp_body,
        grid_spec=grid_spec,
        out_shape=jax.ShapeDtypeStruct((num_tiles * TILE, HIDDEN), jnp.float32),
    )(tile_expert, n_valid, x_pad,
      gate_up_proj, gate_up_proj_bias.reshape(NUM_EXPERTS, 1, 2 * D_FF),
      down_proj, down_proj_bias.reshape(NUM_EXPERTS, 1, HIDDEN),
      sel_even, sel_odd)


def _sc_gather(hs, pos_e, pos_o, p):
    n, h = hs.shape
    nw = 32
    nt = n // nw
    mesh = plsc.VectorSubcoreMesh(core_axis_name="c", subcore_axis_name="s")

    @functools.partial(
        pl.kernel, mesh=mesh,
        out_type=jax.ShapeDtypeStruct((p, h), jnp.float32),
        scratch_types=[
            pltpu.VMEM((nt,), jnp.int32),
            pltpu.VMEM((nt,), jnp.int32),
            pltpu.VMEM((nt, h), jnp.float32),
        ],
    )
    def k(hs_hbm, pe_hbm, po_hbm, out_hbm, ide_v, ido_v, rows_v):
        wid = lax.axis_index("s") * 2 + lax.axis_index("c")
        base_t = wid * nt
        pltpu.sync_copy(pe_hbm.at[pl.ds(base_t, nt)], ide_v)
        pltpu.sync_copy(po_hbm.at[pl.ds(base_t, nt)], ido_v)
        pltpu.sync_copy(hs_hbm.at[pl.ds(base_t, nt)], rows_v)
        pltpu.sync_copy(rows_v, out_hbm.at[ide_v])
        pltpu.sync_copy(rows_v, out_hbm.at[ido_v])

    return k(hs, pos_e, pos_o)


def _sc_combine(y_pad, pos, rw_sel):
    p, h = y_pad.shape
    r = pos.shape[0]
    n = r // TOP_K
    nw = 32
    tpw = n // nw
    half = tpw // 2
    mesh = plsc.VectorSubcoreMesh(core_axis_name="c", subcore_axis_name="s")

    @functools.partial(
        pl.kernel, mesh=mesh,
        out_type=jax.ShapeDtypeStruct((n, h), jnp.float32),
        scratch_types=[
            pltpu.VMEM((2 * half,), jnp.int32),
            pltpu.VMEM((2 * half, 16), jnp.float32),
            pltpu.VMEM((2 * half, h), jnp.float32),
            pltpu.VMEM((half, h), jnp.float32),
        ],
    )
    def k(y_hbm, pos_hbm, rws_hbm, out_hbm, pos_v, rws_v, rows_v, out_v):
        wid = lax.axis_index("s") * 2 + lax.axis_index("c")
        base_t = wid * tpw
        for hh in range(2):
            base_row = base_t * TOP_K + hh * 2 * half
            pltpu.sync_copy(pos_hbm.at[pl.ds(base_row, 2 * half)], pos_v)
            pltpu.sync_copy(rws_hbm.at[pl.ds(base_row, 2 * half)], rws_v)
            pltpu.sync_copy(y_hbm.at[pos_v], rows_v)

            def token_body(j, _):
                w0 = rws_v[2 * j, :]
                w1 = rws_v[2 * j + 1, :]
                for cc in range(h // 16):
                    a = rows_v[2 * j, pl.ds(cc * 16, 16)]
                    bb = rows_v[2 * j + 1, pl.ds(cc * 16, 16)]
                    out_v[j, pl.ds(cc * 16, 16)] = w0 * a + w1 * bb
                return 0

            lax.fori_loop(0, half, token_body, 0)
            pltpu.sync_copy(out_v, out_hbm.at[pl.ds(base_t + hh * half, half)])

    return k(y_pad, pos, rw_sel)


def _lane_shift_add(x, width):
    k = 1
    while k < width:
        x = x + jnp.pad(x, ((0, 0), (k, 0)))[:, :width]
        k *= 2
    return x


def _meta_body(sel_ref, pos_ref, te_ref, u_ref):
    nb, bl = sel_ref.shape
    ne = NUM_EXPERTS
    nt = te_ref.shape[1]

    acc = jnp.zeros((ne, 1), jnp.float32)
    ohs = []
    ranks = []
    ee = lax.broadcasted_iota(jnp.int32, (ne, bl), 0)
    r0 = lax.broadcasted_iota(jnp.int32, (bl, bl), 0)
    c0 = lax.broadcasted_iota(jnp.int32, (bl, bl), 1)
    uinc = jnp.where(r0 <= c0, 1.0, 0.0).astype(jnp.bfloat16)
    for bb in range(nb):
        selb = sel_ref[bb:bb + 1, :]
        oh = jnp.where(selb == ee, 1.0, 0.0)
        csum = lax.dot_general(oh.astype(jnp.bfloat16), uinc,
                               (((1,), (0,)), ((), ())),
                               preferred_element_type=jnp.float32)
        rb = jnp.sum(oh * (csum - 1.0 + acc), axis=0, keepdims=True)
        ohs.append(oh)
        ranks.append(rb)
        acc = acc + jnp.sum(oh, axis=1, keepdims=True)

    counts = acc
    pc = jnp.floor((counts + (TILE - 1.0)) * (1.0 / TILE)) * TILE
    inc = pc
    k = 1
    while k < ne:
        inc = inc + jnp.pad(inc, ((k, 0), (0, 0)))[:ne, :]
        k *= 2
    poff = inc - pc
    u = jnp.sum(pc)

    for bb in range(nb):
        posb = ranks[bb] + jnp.sum(ohs[bb] * poff, axis=0, keepdims=True)
        pos_ref[bb, :] = posb.reshape(bl).astype(jnp.int32)

    starts = (lax.broadcasted_iota(jnp.int32, (1, nt), 1) * TILE).astype(jnp.float32)
    te = jnp.sum(jnp.where(poff <= starts, 1.0, 0.0), axis=0, keepdims=True) - 1.0
    lv_start = jnp.floor((u - 1.0) * (1.0 / TILE)) * TILE
    te_lv = jnp.sum(jnp.where(poff <= lv_start, 1.0, 0.0)) - 1.0
    te = jnp.where(starts < u, te, te_lv)
    te_ref[...] = te.astype(jnp.int32)
    u_ref[...] = jnp.full((1, te_ref.shape[1]), u, jnp.float32).astype(jnp.int32)


def _meta(sel2d, num_tiles):
    nb, bl = sel2d.shape
    return pl.pallas_call(
        _meta_body,
        out_shape=[
            jax.ShapeDtypeStruct((nb, bl), jnp.int32),
            jax.ShapeDtypeStruct((1, num_tiles), jnp.int32),
            jax.ShapeDtypeStruct((1, num_tiles), jnp.int32),
        ],
    )(sel2d)


def kernel(hidden_states, router_indices, routing_weights, gate_up_proj,
           gate_up_proj_bias, down_proj, down_proj_bias):
    b, s, h = hidden_states.shape
    n = b * s
    r = n * TOP_K
    num_tiles = r // TILE + NUM_EXPERTS
    p = num_tiles * TILE

    hs = hidden_states.reshape(n, h)
    sel = router_indices.reshape(-1).astype(jnp.int32)

    pos2d, te2d, u2d = _meta(sel.reshape(-1, 128), num_tiles)
    pos = pos2d.reshape(r)
    te = te2d[0]
    u = u2d[0, :1]

    x_pad = _sc_gather(hs, pos[0::2], pos[1::2], p)

    y_pad = _grouped_mlp(x_pad, te, u, gate_up_proj, gate_up_proj_bias,
                         down_proj, down_proj_bias, num_tiles)

    rw_sel = jnp.take_along_axis(routing_weights, router_indices,
                                 axis=1).reshape(-1)
    rw_b = jnp.broadcast_to(rw_sel[:, None], (r, 16))
    out = _sc_combine(y_pad, pos, rw_b)
    return out.reshape(b, s, h)


# speedup vs baseline: 1.0601x; 1.0601x over previous
import functools

import jax
import jax.numpy as jnp
from jax import lax
from jax.experimental import pallas as pl
from jax.experimental.pallas import tpu as pltpu
from jax.experimental.pallas import tpu_sc as plsc

NUM_EXPERTS = 64
HIDDEN = 768
D_FF = 768
ALPHA = 1.702
LIMIT = 7.0
TOP_K = 2

TILE = 64
SUB = 8


def _mlp_body(te_ref, nv_ref, x_ref, guw_ref, gub_ref, dnw_ref, dnb_ref,
              se_ref, so_ref, y_ref, wbuf, bbuf, dbuf, ebuf, sems, slot_ref):
    t = pl.program_id(0)
    nt = pl.num_programs(0)

    def fetch(e, s, wait):
        cps = [
            pltpu.make_async_copy(guw_ref.at[e], wbuf.at[s], sems.at[s, 0]),
            pltpu.make_async_copy(gub_ref.at[e], bbuf.at[s], sems.at[s, 1]),
            pltpu.make_async_copy(dnw_ref.at[e], dbuf.at[s], sems.at[s, 2]),
            pltpu.make_async_copy(dnb_ref.at[e], ebuf.at[s], sems.at[s, 3]),
        ]
        for cp in cps:
            if wait:
                cp.wait()
            else:
                cp.start()

    nj = nt * SUB
    for k in range(SUB):
        j = t * SUB + k
        cur = te_ref[j]
        prev = jnp.where(j > 0, te_ref[jnp.maximum(j - 1, 0)], -1)
        nxt = jnp.where(j + 1 < nj, te_ref[jnp.minimum(j + 1, nj - 1)], cur)
        changed_in = cur != prev

        if k == 0:
            @pl.when(t == 0)
            def _():
                slot_ref[0] = 0
                fetch(cur, 0, wait=False)

        @pl.when(changed_in & (j > 0))
        def _():
            slot_ref[0] = 1 - slot_ref[0]

        s = slot_ref[0]

        @pl.when(changed_in)
        def _():
            fetch(cur, s, wait=True)

        @pl.when(nxt != cur)
        def _():
            fetch(nxt, 1 - s, wait=False)

        @pl.when(j * TILE < nv_ref[0])
        def _():
            x = x_ref[k * TILE:(k + 1) * TILE, :]
            gu = lax.dot_general(x, wbuf[s], (((1,), (0,)), ((), ())),
                                 preferred_element_type=jnp.float32)
            a = (gu + bbuf[s]).astype(jnp.bfloat16)
            gate = lax.dot_general(a, se_ref[...], (((1,), (0,)), ((), ())),
                                   preferred_element_type=jnp.float32)
            up = lax.dot_general(a, so_ref[...], (((1,), (0,)), ((), ())),
                                 preferred_element_type=jnp.float32)
            gate = jnp.minimum(gate, LIMIT)
            up = jnp.clip(up, -LIMIT, LIMIT)
            glu = gate * jax.nn.sigmoid(gate * ALPHA)
            h = (up + 1.0) * glu
            y = lax.dot_general(h, dbuf[s], (((1,), (0,)), ((), ())),
                                preferred_element_type=jnp.float32)
            y_ref[k * TILE:(k + 1) * TILE, :] = y + ebuf[s]


def _grouped_mlp(x_pad, tile_expert, n_valid, gate_up_proj, gate_up_proj_bias,
                 down_proj, down_proj_bias, num_tiles):
    hbm = pl.BlockSpec(memory_space=pltpu.MemorySpace.HBM)
    grid_spec = pltpu.PrefetchScalarGridSpec(
        num_scalar_prefetch=2,
        grid=(num_tiles // SUB,),
        in_specs=[
            pl.BlockSpec((SUB * TILE, HIDDEN), lambda t, te, nv: (t, 0)),
            hbm, hbm, hbm, hbm,
            pl.BlockSpec((2 * D_FF, D_FF), lambda t, te, nv: (0, 0)),
            pl.BlockSpec((2 * D_FF, D_FF), lambda t, te, nv: (0, 0)),
        ],
        out_specs=pl.BlockSpec((SUB * TILE, HIDDEN), lambda t, te, nv: (t, 0)),
        scratch_shapes=[
            pltpu.VMEM((2, HIDDEN, 2 * D_FF), jnp.float32),
            pltpu.VMEM((2, 1, 2 * D_FF), jnp.float32),
            pltpu.VMEM((2, D_FF, HIDDEN), jnp.float32),
            pltpu.VMEM((2, 1, HIDDEN), jnp.float32),
            pltpu.SemaphoreType.DMA((2, 4)),
            pltpu.SMEM((1,), jnp.int32),
        ],
    )
    ff = jnp.arange(D_FF, dtype=jnp.int32)
    col = jnp.arange(2 * D_FF, dtype=jnp.int32)
    sel_even = (col[:, None] == 2 * ff[None, :]).astype(jnp.bfloat16)
    sel_odd = (col[:, None] == 2 * ff[None, :] + 1).astype(jnp.bfloat16)
    return pl.pallas_call(
        _mlp_body,
        grid_spec=grid_spec,
        out_shape=jax.ShapeDtypeStruct((num_tiles * TILE, HIDDEN), jnp.float32),
    )(tile_expert, n_valid, x_pad,
      gate_up_proj, gate_up_proj_bias.reshape(NUM_EXPERTS, 1, 2 * D_FF),
      down_proj, down_proj_bias.reshape(NUM_EXPERTS, 1, HIDDEN),
      sel_even, sel_odd)


def _sc_gather(hs, pos_e, pos_o, p):
    n, h = hs.shape
    nw = 32
    nt = n // nw
    mesh = plsc.VectorSubcoreMesh(core_axis_name="c", subcore_axis_name="s")

    @functools.partial(
        pl.kernel, mesh=mesh,
        out_type=jax.ShapeDtypeStruct((p, h), jnp.float32),
        scratch_types=[
            pltpu.VMEM((nt,), jnp.int32),
            pltpu.VMEM((nt,), jnp.int32),
            pltpu.VMEM((nt, h), jnp.float32),
        ],
    )
    def k(hs_hbm, pe_hbm, po_hbm, out_hbm, ide_v, ido_v, rows_v):
        wid = lax.axis_index("s") * 2 + lax.axis_index("c")
        base_t = wid * nt
        pltpu.sync_copy(pe_hbm.at[pl.ds(base_t, nt)], ide_v)
        pltpu.sync_copy(po_hbm.at[pl.ds(base_t, nt)], ido_v)
        pltpu.sync_copy(hs_hbm.at[pl.ds(base_t, nt)], rows_v)
        pltpu.sync_copy(rows_v, out_hbm.at[ide_v])
        pltpu.sync_copy(rows_v, out_hbm.at[ido_v])

    return k(hs, pos_e, pos_o)


def _sc_combine(y_pad, pos, rw_sel):
    p, h = y_pad.shape
    r = pos.shape[0]
    n = r // TOP_K
    nw = 32
    tpw = n // nw
    half = tpw // 2
    mesh = plsc.VectorSubcoreMesh(core_axis_name="c", subcore_axis_name="s")

    @functools.partial(
        pl.kernel, mesh=mesh,
        out_type=jax.ShapeDtypeStruct((n, h), jnp.float32),
        scratch_types=[
            pltpu.VMEM((2 * half,), jnp.int32),
            pltpu.VMEM((2 * half, 16), jnp.float32),
            pltpu.VMEM((2 * half, h), jnp.float32),
            pltpu.VMEM((half, h), jnp.float32),
        ],
    )
    def k(y_hbm, pos_hbm, rws_hbm, out_hbm, pos_v, rws_v, rows_v, out_v):
        wid = lax.axis_index("s") * 2 + lax.axis_index("c")
        base_t = wid * tpw
        for hh in range(2):
            base_row = base_t * TOP_K + hh * 2 * half
            pltpu.sync_copy(pos_hbm.at[pl.ds(base_row, 2 * half)], pos_v)
            pltpu.sync_copy(rws_hbm.at[pl.ds(base_row, 2 * half)], rws_v)
            pltpu.sync_copy(y_hbm.at[pos_v], rows_v)

            def token_body(j, _):
                w0 = rws_v[2 * j, :]
                w1 = rws_v[2 * j + 1, :]
                for cc in range(h // 16):
                    a = rows_v[2 * j, pl.ds(cc * 16, 16)]
                    bb = rows_v[2 * j + 1, pl.ds(cc * 16, 16)]
                    out_v[j, pl.ds(cc * 16, 16)] = w0 * a + w1 * bb
                return 0

            lax.fori_loop(0, half, token_body, 0)
            pltpu.sync_copy(out_v, out_hbm.at[pl.ds(base_t + hh * half, half)])

    return k(y_pad, pos, rw_sel)


def _lane_shift_add(x, width):
    k = 1
    while k < width:
        x = x + jnp.pad(x, ((0, 0), (k, 0)))[:, :width]
        k *= 2
    return x


def _meta_body(sel_ref, pos_ref, te_ref, u_ref):
    nb, bl = sel_ref.shape
    ne = NUM_EXPERTS
    nt = te_ref.shape[1]

    acc = jnp.zeros((ne, 1), jnp.float32)
    ohs = []
    ranks = []
    ee = lax.broadcasted_iota(jnp.int32, (ne, bl), 0)
    r0 = lax.broadcasted_iota(jnp.int32, (bl, bl), 0)
    c0 = lax.broadcasted_iota(jnp.int32, (bl, bl), 1)
    uinc = jnp.where(r0 <= c0, 1.0, 0.0).astype(jnp.bfloat16)
    for bb in range(nb):
        selb = sel_ref[bb:bb + 1, :]
        oh = jnp.where(selb == ee, 1.0, 0.0)
        csum = lax.dot_general(oh.astype(jnp.bfloat16), uinc,
                               (((1,), (0,)), ((), ())),
                               preferred_element_type=jnp.float32)
        rb = jnp.sum(oh * (csum - 1.0 + acc), axis=0, keepdims=True)
        ohs.append(oh)
        ranks.append(rb)
        acc = acc + jnp.sum(oh, axis=1, keepdims=True)

    counts = acc
    pc = jnp.floor((counts + (TILE - 1.0)) * (1.0 / TILE)) * TILE
    inc = pc
    k = 1
    while k < ne:
        inc = inc + jnp.pad(inc, ((k, 0), (0, 0)))[:ne, :]
        k *= 2
    poff = inc - pc
    u = jnp.sum(pc)

    for bb in range(nb):
        posb = ranks[bb] + jnp.sum(ohs[bb] * poff, axis=0, keepdims=True)
        pos_ref[bb, :] = posb.reshape(bl).astype(jnp.int32)

    starts = (lax.broadcasted_iota(jnp.int32, (1, nt), 1) * TILE).astype(jnp.float32)
    te = jnp.sum(jnp.where(poff <= starts, 1.0, 0.0), axis=0, keepdims=True) - 1.0
    lv_start = jnp.floor((u - 1.0) * (1.0 / TILE)) * TILE
    te_lv = jnp.sum(jnp.where(poff <= lv_start, 1.0, 0.0)) - 1.0
    te = jnp.where(starts < u, te, te_lv)
    te_ref[...] = te.astype(jnp.int32)
    u_ref[...] = jnp.full((1, te_ref.shape[1]), u, jnp.float32).astype(jnp.int32)


def _meta(sel2d, num_tiles):
    nb, bl = sel2d.shape
    return pl.pallas_call(
        _meta_body,
        out_shape=[
            jax.ShapeDtypeStruct((nb, bl), jnp.int32),
            jax.ShapeDtypeStruct((1, num_tiles), jnp.int32),
            jax.ShapeDtypeStruct((1, num_tiles), jnp.int32),
        ],
    )(sel2d)


def kernel(hidden_states, router_indices, routing_weights, gate_up_proj,
           gate_up_proj_bias, down_proj, down_proj_bias):
    b, s, h = hidden_states.shape
    n = b * s
    r = n * TOP_K
    num_tiles = r // TILE + NUM_EXPERTS
    p = num_tiles * TILE

    hs = hidden_states.reshape(n, h)
    sel = router_indices.reshape(-1).astype(jnp.int32)

    pos2d, te2d, u2d = _meta(sel.reshape(-1, 128), num_tiles)
    pos = pos2d.reshape(r)
    te = te2d[0]
    u = u2d[0, :1]

    x_pad = _sc_gather(hs, pos[0::2], pos[1::2], p)

    y_pad = _grouped_mlp(x_pad, te, u, gate_up_proj, gate_up_proj_bias,
                         down_proj, down_proj_bias, num_tiles)

    rw_sel = jnp.take_along_axis(routing_weights, router_indices,
                                 axis=1).reshape(-1)
    rw_b = jnp.broadcast_to(rw_sel[:, None], (r, 16))
    out = _sc_combine(y_pad, pos, rw_b)
    return out.reshape(b, s, h)


# speedup vs baseline: 1.2223x; 1.1530x over previous
import functools

import jax
import jax.numpy as jnp
from jax import lax
from jax.experimental import pallas as pl
from jax.experimental.pallas import tpu as pltpu
from jax.experimental.pallas import tpu_sc as plsc

NUM_EXPERTS = 64
HIDDEN = 768
D_FF = 768
ALPHA = 1.702
LIMIT = 7.0
TOP_K = 2

TILE = 64
SUB = 8


def _mlp_body(te_ref, nv_ref, x_ref, guw_ref, gub_ref, dnw_ref, dnb_ref,
              se_ref, so_ref, y_ref, wbuf, bbuf, dbuf, ebuf, sems, slot_ref):
    t = pl.program_id(0)
    nt = pl.num_programs(0)

    def fetch(e, s, wait, part=None):
        cps = [
            pltpu.make_async_copy(guw_ref.at[e], wbuf.at[s], sems.at[s, 0]),
            pltpu.make_async_copy(gub_ref.at[e], bbuf.at[s], sems.at[s, 1]),
            pltpu.make_async_copy(dnw_ref.at[e], dbuf.at[s], sems.at[s, 2]),
            pltpu.make_async_copy(dnb_ref.at[e], ebuf.at[s], sems.at[s, 3]),
        ]
        if part is not None:
            cps = [cps[i] for i in part]
        for cp in cps:
            if wait:
                cp.wait()
            else:
                cp.start()

    nj = nt * SUB
    for k in range(SUB):
        j = t * SUB + k
        cur = te_ref[j]
        prev = jnp.where(j > 0, te_ref[jnp.maximum(j - 1, 0)], -1)
        nxt = jnp.where(j + 1 < nj, te_ref[jnp.minimum(j + 1, nj - 1)], cur)
        changed_in = cur != prev

        if k == 0:
            @pl.when(t == 0)
            def _():
                slot_ref[0] = 0
                fetch(cur, 0, wait=False)

        @pl.when(changed_in & (j > 0))
        def _():
            slot_ref[0] = 1 - slot_ref[0]

        s = slot_ref[0]

        @pl.when(nxt != cur)
        def _():
            fetch(nxt, 1 - s, wait=False)

        @pl.when(j * TILE < nv_ref[0])
        def _():
            x = x_ref[k * TILE:(k + 1) * TILE, :]

            @pl.when(changed_in)
            def _():
                fetch(cur, s, wait=True, part=(0, 1))

            gu = lax.dot_general(x, wbuf[s], (((1,), (0,)), ((), ())),
                                 preferred_element_type=jnp.float32)
            a = (gu + bbuf[s]).astype(jnp.bfloat16)
            gate = lax.dot_general(a, se_ref[...], (((1,), (0,)), ((), ())),
                                   preferred_element_type=jnp.float32)
            up = lax.dot_general(a, so_ref[...], (((1,), (0,)), ((), ())),
                                 preferred_element_type=jnp.float32)
            gate = jnp.minimum(gate, LIMIT)
            up = jnp.clip(up, -LIMIT, LIMIT)
            glu = gate * jax.nn.sigmoid(gate * ALPHA)
            h = (up + 1.0) * glu

            @pl.when(changed_in)
            def _():
                fetch(cur, s, wait=True, part=(2, 3))

            y = lax.dot_general(h, dbuf[s], (((1,), (0,)), ((), ())),
                                preferred_element_type=jnp.float32)
            y_ref[k * TILE:(k + 1) * TILE, :] = y + ebuf[s]

        @pl.when((j * TILE >= nv_ref[0]) & changed_in)
        def _():
            fetch(cur, s, wait=True)


def _grouped_mlp(x_pad, tile_expert, n_valid, gate_up_proj, gate_up_proj_bias,
                 down_proj, down_proj_bias, num_tiles):
    hbm = pl.BlockSpec(memory_space=pltpu.MemorySpace.HBM)
    grid_spec = pltpu.PrefetchScalarGridSpec(
        num_scalar_prefetch=2,
        grid=(num_tiles // SUB,),
        in_specs=[
            pl.BlockSpec((SUB * TILE, HIDDEN), lambda t, te, nv: (t, 0)),
            hbm, hbm, hbm, hbm,
            pl.BlockSpec((2 * D_FF, D_FF), lambda t, te, nv: (0, 0)),
            pl.BlockSpec((2 * D_FF, D_FF), lambda t, te, nv: (0, 0)),
        ],
        out_specs=pl.BlockSpec((SUB * TILE, HIDDEN), lambda t, te, nv: (t, 0)),
        scratch_shapes=[
            pltpu.VMEM((2, HIDDEN, 2 * D_FF), jnp.float32),
            pltpu.VMEM((2, 1, 2 * D_FF), jnp.float32),
            pltpu.VMEM((2, D_FF, HIDDEN), jnp.float32),
            pltpu.VMEM((2, 1, HIDDEN), jnp.float32),
            pltpu.SemaphoreType.DMA((2, 4)),
            pltpu.SMEM((1,), jnp.int32),
        ],
    )
    ff = jnp.arange(D_FF, dtype=jnp.int32)
    col = jnp.arange(2 * D_FF, dtype=jnp.int32)
    sel_even = (col[:, None] == 2 * ff[None, :]).astype(jnp.bfloat16)
    sel_odd = (col[:, None] == 2 * ff[None, :] + 1).astype(jnp.bfloat16)
    return pl.pallas_call(
        _mlp_body,
        grid_spec=grid_spec,
        out_shape=jax.ShapeDtypeStruct((num_tiles * TILE, HIDDEN), jnp.float32),
    )(tile_expert, n_valid, x_pad,
      gate_up_proj, gate_up_proj_bias.reshape(NUM_EXPERTS, 1, 2 * D_FF),
      down_proj, down_proj_bias.reshape(NUM_EXPERTS, 1, HIDDEN),
      sel_even, sel_odd)


def _sc_gather(hs, pos_e, pos_o, p):
    n, h = hs.shape
    nw = 32
    nt = n // nw
    mesh = plsc.VectorSubcoreMesh(core_axis_name="c", subcore_axis_name="s")

    @functools.partial(
        pl.kernel, mesh=mesh,
        out_type=jax.ShapeDtypeStruct((p, h), jnp.float32),
        scratch_types=[
            pltpu.VMEM((nt,), jnp.int32),
            pltpu.VMEM((nt,), jnp.int32),
            pltpu.VMEM((nt, h), jnp.float32),
        ],
    )
    def k(hs_hbm, pe_hbm, po_hbm, out_hbm, ide_v, ido_v, rows_v):
        wid = lax.axis_index("s") * 2 + lax.axis_index("c")
        base_t = wid * nt
        pltpu.sync_copy(pe_hbm.at[pl.ds(base_t, nt)], ide_v)
        pltpu.sync_copy(po_hbm.at[pl.ds(base_t, nt)], ido_v)
        pltpu.sync_copy(hs_hbm.at[pl.ds(base_t, nt)], rows_v)
        pltpu.sync_copy(rows_v, out_hbm.at[ide_v])
        pltpu.sync_copy(rows_v, out_hbm.at[ido_v])

    return k(hs, pos_e, pos_o)


def _sc_combine(y_pad, pos, rw_sel):
    p, h = y_pad.shape
    r = pos.shape[0]
    n = r // TOP_K
    nw = 32
    tpw = n // nw
    half = tpw // 2
    mesh = plsc.VectorSubcoreMesh(core_axis_name="c", subcore_axis_name="s")

    @functools.partial(
        pl.kernel, mesh=mesh,
        out_type=jax.ShapeDtypeStruct((n, h), jnp.float32),
        scratch_types=[
            pltpu.VMEM((2 * half,), jnp.int32),
            pltpu.VMEM((2 * half, 16), jnp.float32),
            pltpu.VMEM((2 * half, h), jnp.float32),
            pltpu.VMEM((half, h), jnp.float32),
        ],
    )
    def k(y_hbm, pos_hbm, rws_hbm, out_hbm, pos_v, rws_v, rows_v, out_v):
        wid = lax.axis_index("s") * 2 + lax.axis_index("c")
        base_t = wid * tpw
        for hh in range(2):
            base_row = base_t * TOP_K + hh * 2 * half
            pltpu.sync_copy(pos_hbm.at[pl.ds(base_row, 2 * half)], pos_v)
            pltpu.sync_copy(rws_hbm.at[pl.ds(base_row, 2 * half)], rws_v)
            pltpu.sync_copy(y_hbm.at[pos_v], rows_v)

            def token_body(j, _):
                w0 = rws_v[2 * j, :]
                w1 = rws_v[2 * j + 1, :]
                for cc in range(h // 16):
                    a = rows_v[2 * j, pl.ds(cc * 16, 16)]
                    bb = rows_v[2 * j + 1, pl.ds(cc * 16, 16)]
                    out_v[j, pl.ds(cc * 16, 16)] = w0 * a + w1 * bb
                return 0

            lax.fori_loop(0, half, token_body, 0)
            pltpu.sync_copy(out_v, out_hbm.at[pl.ds(base_t + hh * half, half)])

    return k(y_pad, pos, rw_sel)


def _lane_shift_add(x, width):
    k = 1
    while k < width:
        x = x + jnp.pad(x, ((0, 0), (k, 0)))[:, :width]
        k *= 2
    return x


def _meta_body(sel_ref, pos_ref, te_ref, u_ref):
    nb, bl = sel_ref.shape
    ne = NUM_EXPERTS
    nt = te_ref.shape[1]

    acc = jnp.zeros((ne, 1), jnp.float32)
    ohs = []
    ranks = []
    ee = lax.broadcasted_iota(jnp.int32, (ne, bl), 0)
    r0 = lax.broadcasted_iota(jnp.int32, (bl, bl), 0)
    c0 = lax.broadcasted_iota(jnp.int32, (bl, bl), 1)
    uinc = jnp.where(r0 <= c0, 1.0, 0.0).astype(jnp.bfloat16)
    for bb in range(nb):
        selb = sel_ref[bb:bb + 1, :]
        oh = jnp.where(selb == ee, 1.0, 0.0)
        csum = lax.dot_general(oh.astype(jnp.bfloat16), uinc,
                               (((1,), (0,)), ((), ())),
                               preferred_element_type=jnp.float32)
        rb = jnp.sum(oh * (csum - 1.0 + acc), axis=0, keepdims=True)
        ohs.append(oh)
        ranks.append(rb)
        acc = acc + jnp.sum(oh, axis=1, keepdims=True)

    counts = acc
    pc = jnp.floor((counts + (TILE - 1.0)) * (1.0 / TILE)) * TILE
    inc = pc
    k = 1
    while k < ne:
        inc = inc + jnp.pad(inc, ((k, 0), (0, 0)))[:ne, :]
        k *= 2
    poff = inc - pc
    u = jnp.sum(pc)

    for bb in range(nb):
        posb = ranks[bb] + jnp.sum(ohs[bb] * poff, axis=0, keepdims=True)
        pos_ref[bb, :] = posb.reshape(bl).astype(jnp.int32)

    starts = (lax.broadcasted_iota(jnp.int32, (1, nt), 1) * TILE).astype(jnp.float32)
    te = jnp.sum(jnp.where(poff <= starts, 1.0, 0.0), axis=0, keepdims=True) - 1.0
    lv_start = jnp.floor((u - 1.0) * (1.0 / TILE)) * TILE
    te_lv = jnp.sum(jnp.where(poff <= lv_start, 1.0, 0.0)) - 1.0
    te = jnp.where(starts < u, te, te_lv)
    te_ref[...] = te.astype(jnp.int32)
    u_ref[...] = jnp.full((1, te_ref.shape[1]), u, jnp.float32).astype(jnp.int32)


def _meta(sel2d, num_tiles):
    nb, bl = sel2d.shape
    return pl.pallas_call(
        _meta_body,
        out_shape=[
            jax.ShapeDtypeStruct((nb, bl), jnp.int32),
            jax.ShapeDtypeStruct((1, num_tiles), jnp.int32),
            jax.ShapeDtypeStruct((1, num_tiles), jnp.int32),
        ],
    )(sel2d)


def kernel(hidden_states, router_indices, routing_weights, gate_up_proj,
           gate_up_proj_bias, down_proj, down_proj_bias):
    b, s, h = hidden_states.shape
    n = b * s
    r = n * TOP_K
    num_tiles = r // TILE + NUM_EXPERTS
    p = num_tiles * TILE

    hs = hidden_states.reshape(n, h)
    sel = router_indices.reshape(-1).astype(jnp.int32)

    pos2d, te2d, u2d = _meta(sel.reshape(-1, 128), num_tiles)
    pos = pos2d.reshape(r)
    te = te2d[0]
    u = u2d[0, :1]

    x_pad = _sc_gather(hs, pos[0::2], pos[1::2], p)

    y_pad = _grouped_mlp(x_pad, te, u, gate_up_proj, gate_up_proj_bias,
                         down_proj, down_proj_bias, num_tiles)

    rw_sel = jnp.take_along_axis(routing_weights, router_indices,
                                 axis=1).reshape(-1)
    rw_b = jnp.broadcast_to(rw_sel[:, None], (r, 16))
    out = _sc_combine(y_pad, pos, rw_b)
    return out.reshape(b, s, h)
